# baseline (speedup 1.0000x reference)
import functools

import jax
import jax.numpy as jnp
from jax import lax
from jax.experimental import pallas as pl
from jax.experimental.pallas import tpu as pltpu
from jax.experimental.pallas import tpu_sc as plsc

RES = (512, 264, 16)
NLVL = 3
NC, NS, L = 2, 16, 16
NW = NC * NS
B = 1024
ROWS_PER_DMA = 256
NDMA = B // ROWS_PER_DMA

H = (131584, 35072, 144)
_BQ = (2056, 2192)


def _pair_table_small(emb, r):
    s = (r + 1) * (r + 1)
    h = (4 * s) // 8
    flat = emb.reshape(-1)
    return jnp.concatenate(
        [flat[: 8 * h].reshape(h, 8), flat[4 : 4 + 8 * h].reshape(h, 8)], axis=0)


def _table_body(a_ref, b_ref, o_ref):
    a = a_ref[...]

    @pl.when(pl.program_id(0) == 0)
    def _():
        o_ref[...] = a

    @pl.when(pl.program_id(0) == 1)
    def _():
        b = b_ref[...]
        a1 = jnp.concatenate([a[1:], b[:1]], axis=0)
        o_ref[...] = jnp.concatenate([a[:, 4:], a1[:, :4]], axis=1)


def _pair_table_big(emb, h, bq):
    flat = emb.reshape(-1)
    fv = flat.shape[0] // 128
    wide = flat[: fv * 128].reshape(fv, 128)
    qh = h // 16
    nq = qh // bq
    out = pl.pallas_call(
        _table_body,
        grid=(2, nq),
        in_specs=[
            pl.BlockSpec((bq, 128), lambda h2, q: (q, 0)),
            pl.BlockSpec((bq, 128), lambda h2, q: (q + 1, 0)),
        ],
        out_specs=pl.BlockSpec((bq, 128), lambda h2, q: (h2 * (h // 16 // bq) + q, 0)),
        out_shape=jax.ShapeDtypeStruct((2 * qh, 128), jnp.float32),
    )(wide, wide)
    return out.reshape(2 * h, 8)


def _sc_gather(xflat, n, lo, npts, t0, t1, t2):
    pts_per_w = npts // NW
    nchunk = pts_per_w // B

    mesh = plsc.VectorSubcoreMesh(
        core_axis_name="c", subcore_axis_name="s", num_cores=NC, num_subcores=NS
    )
    scratch = (
        [pltpu.VMEM((B,), jnp.float32),
         pltpu.VMEM((B,), jnp.float32)]
        + [pltpu.VMEM((B,), jnp.int32) for _ in range(2 * NLVL)]
        + [pltpu.VMEM((B, 8), jnp.float32) for _ in range(2 * NLVL)]
        + [pltpu.SemaphoreType.DMA for _ in range(2 * NLVL)]
    )

    @functools.partial(
        pl.kernel,
        out_type=tuple(jax.ShapeDtypeStruct((npts, 8), jnp.float32)
                       for _ in range(2 * NLVL)),
        mesh=mesh,
        scratch_types=scratch,
        compiler_params=pltpu.CompilerParams(use_tc_tiling_on_sc=False),
    )
    def k(x_hbm, t0_hbm, t1_hbm, t2_hbm, o0u, o0d, o1u, o1d, o2u, o2d,
          u_ref, v_ref, i0u, i0d, i1u, i1d, i2u, i2d,
          r0u, r0d, r1u, r1d, r2u, r2d, s0u, s0d, s1u, s1d, s2u, s2d):
        t_hbm = (t0_hbm, t0_hbm, t1_hbm, t1_hbm, t2_hbm, t2_hbm)
        out_hbm = (o0u, o0d, o1u, o1d, o2u, o2d)
        idx_refs = (i0u, i0d, i1u, i1d, i2u, i2d)
        row_refs = (r0u, r0d, r1u, r1d, r2u, r2d)
        sems = (s0u, s0d, s1u, s1d, s2u, s2d)
        wid = lax.axis_index("s") * NC + lax.axis_index("c")
        groups = B // L

        def chunk_body(ci, carry):
            base = wid * pts_per_w + ci * B
            pltpu.sync_copy(x_hbm.at[pl.ds(n + lo + base, B)], u_ref)
            pltpu.sync_copy(x_hbm.at[pl.ds(2 * n + lo + base, B)], v_ref)

            def idx_body(j, c):
                sl = pl.ds(j * L, L)
                u = u_ref[sl]
                v = v_ref[sl]
                for lvl, r in enumerate(RES):
                    x0 = (u * r).astype(jnp.int32)
                    y0 = (v * r).astype(jnp.int32)
                    p = y0 * r + x0
                    up = (p >> 1) + (p & 1) * H[lvl]
                    idx_refs[2 * lvl][sl] = up
                    idx_refs[2 * lvl + 1][sl] = up + (r // 2)
                return c

            lax.fori_loop(0, groups, idx_body, 0)

            dmas = []
            for st in range(2 * NLVL):
                for g in range(NDMA):
                    dmas.append(pltpu.async_copy(
                        t_hbm[st].at[idx_refs[st].at[pl.ds(g * ROWS_PER_DMA, ROWS_PER_DMA)]],
                        row_refs[st].at[pl.ds(g * ROWS_PER_DMA, ROWS_PER_DMA)],
                        sems[st],
                    ))
            for st in range(2 * NLVL):
                for g in range(NDMA):
                    dmas[st * NDMA + g].wait()
                pltpu.sync_copy(row_refs[st], out_hbm[st].at[pl.ds(base, B), :])
            return carry

        lax.fori_loop(0, nchunk, chunk_body, 0)

    return k(xflat, t0, t1, t2)


def _tc_body(u0, d0, u1, d1, u2, d2, u8, v8, i8, eb, g0, g1, g2, gi,
             b0t, w1t, b1r, w2t, b2r, out_ref):
    e = eb[...]
    u64 = jnp.dot(u8[...], e, preferred_element_type=jnp.float32)
    v64 = jnp.dot(v8[...], e, preferred_element_type=jnp.float32)
    cf = jax.lax.broadcasted_iota(jnp.int32, (1, 64), 1) % 8
    is_x1 = cf >= 4
    acc = jnp.dot(i8[...], gi[...], preferred_element_type=jnp.float32)
    ups = (u0, u1, u2)
    dns = (d0, d1, d2)
    gs = (g0, g1, g2)
    for lvl, r in enumerate(RES):
        fx = u64 * r
        wx = fx - jnp.floor(fx)
        fy = v64 * r
        wy = fy - jnp.floor(fy)
        wxs = jnp.where(is_x1, wx, 1.0 - wx)
        t = wxs * (ups[lvl][...] * (1.0 - wy) + dns[lvl][...] * wy)
        acc = acc + jnp.dot(t, gs[lvl][...], preferred_element_type=jnp.float32)
    h1 = jnp.maximum(acc + b0t[...], 0.0)
    outs = []
    for j in range(8):
        hj = h1[:, 64 * j:64 * j + 64]
        h2 = jnp.maximum(
            jnp.dot(hj, w1t[...], preferred_element_type=jnp.float32) + b1r[...], 0.0)
        outs.append(
            jnp.dot(h2, w2t[...], preferred_element_type=jnp.float32) + b2r[...])
    out_ref[...] = jnp.concatenate(outs, axis=1)


def _tc_mlp(c, u8, v8, i8, off, mh, W0, b0, W1, b1, W2, b2):
    mb = 1024
    grid = (mh // mb,)
    eye8 = jnp.eye(8, dtype=jnp.float32)
    eb = jnp.kron(eye8, jnp.ones((1, 8), jnp.float32))
    gi = jnp.kron(eye8, W0[:, 0:1].T)
    gs = []
    for lvl in range(NLVL):
        e8 = jnp.tile(W0[:, 1 + 4 * lvl:5 + 4 * lvl].T, (2, 1))
        gs.append(jnp.kron(eye8, e8))
    b0t = jnp.tile(b0, 8)[None, :]
    row = lambda i: (i, 0)
    rowo = lambda i: (i + off, 0)
    full = lambda i: (0, 0)
    out_dim = W2.shape[0]
    return pl.pallas_call(
        _tc_body,
        grid=grid,
        in_specs=[
            pl.BlockSpec((mb, 64), rowo),
            pl.BlockSpec((mb, 64), rowo),
            pl.BlockSpec((mb, 64), rowo),
            pl.BlockSpec((mb, 64), rowo),
            pl.BlockSpec((mb, 64), rowo),
            pl.BlockSpec((mb, 64), rowo),
            pl.BlockSpec((mb, 8), rowo),
            pl.BlockSpec((mb, 8), rowo),
            pl.BlockSpec((mb, 8), rowo),
            pl.BlockSpec((8, 64), full),
            pl.BlockSpec((64, 512), full),
            pl.BlockSpec((64, 512), full),
            pl.BlockSpec((64, 512), full),
            pl.BlockSpec((8, 512), full),
            pl.BlockSpec((1, 512), full),
            pl.BlockSpec((64, 64), full),
            pl.BlockSpec((1, 64), full),
            pl.BlockSpec((64, out_dim), full),
            pl.BlockSpec((1, out_dim), full),
        ],
        out_specs=pl.BlockSpec((mb, 8 * out_dim), row),
        out_shape=jax.ShapeDtypeStruct((mh, 8 * out_dim), jnp.float32),
    )(*c, u8, v8, i8, eb, gs[0], gs[1], gs[2], gi, b0t,
      W1.T, b1[None, :], W2.T, b2[None, :])


def kernel(x, emb0, emb1, emb2, W0, b0, W1, b1, W2, b2):
    n = x.shape[0]
    nh = n // 2
    xT = x.T
    xflat = xT.reshape(-1)
    tables = [
        _pair_table_big(emb0, H[0], _BQ[0]),
        _pair_table_big(emb1, H[1], _BQ[1]),
        _pair_table_small(emb2, RES[2]),
    ]
    c = _sc_gather(xflat, n, 0, n, *tables)
    m = n // 8
    mh = nh // 8
    u8 = xT[1].reshape(m, 8)
    v8 = xT[2].reshape(m, 8)
    i8 = xT[0].reshape(m, 8)
    packed = [a.reshape(m, 64) for a in c]
    o1 = _tc_mlp(packed, u8, v8, i8, 0, mh, W0, b0, W1, b1, W2, b2)
    o2 = _tc_mlp(packed, u8, v8, i8, mh // 1024, mh, W0, b0, W1, b1, W2, b2)
    return jnp.concatenate([o1, o2], axis=0).reshape(n, W2.shape[0])


# speedup vs baseline: 1.0215x; 1.0215x over previous
import functools

import jax
import jax.numpy as jnp
from jax import lax
from jax.experimental import pallas as pl
from jax.experimental.pallas import tpu as pltpu
from jax.experimental.pallas import tpu_sc as plsc

RES = (512, 264, 16)
NLVL = 3
NC, NS, L = 2, 16, 16
NW = NC * NS
B = 1024
ROWS_PER_DMA = 256
NDMA = B // ROWS_PER_DMA

H = (131584, 35072, 144)
_BQ = (2056, 2192)


def _pair_table_small(emb, r):
    s = (r + 1) * (r + 1)
    h = (4 * s) // 8
    flat = emb.reshape(-1)
    return jnp.concatenate(
        [flat[: 8 * h].reshape(h, 8), flat[4 : 4 + 8 * h].reshape(h, 8)], axis=0)


def _table_body(a_ref, b_ref, o_ref):
    a = a_ref[...]

    @pl.when(pl.program_id(0) == 0)
    def _():
        o_ref[...] = a

    @pl.when(pl.program_id(0) == 1)
    def _():
        b = b_ref[...]
        a1 = jnp.concatenate([a[1:], b[:1]], axis=0)
        o_ref[...] = jnp.concatenate([a[:, 4:], a1[:, :4]], axis=1)


def _pair_table_big(emb, h, bq):
    flat = emb.reshape(-1)
    fv = flat.shape[0] // 128
    wide = flat[: fv * 128].reshape(fv, 128)
    qh = h // 16
    nq = qh // bq
    out = pl.pallas_call(
        _table_body,
        grid=(2, nq),
        in_specs=[
            pl.BlockSpec((bq, 128), lambda h2, q: (q, 0)),
            pl.BlockSpec((bq, 128), lambda h2, q: (q + 1, 0)),
        ],
        out_specs=pl.BlockSpec((bq, 128), lambda h2, q: (h2 * (h // 16 // bq) + q, 0)),
        out_shape=jax.ShapeDtypeStruct((2 * qh, 128), jnp.float32),
    )(wide, wide)
    return out.reshape(2 * h, 8)


def _sc_gather(xflat, n, lo, npts, t0, t1, t2):
    pts_per_w = npts // NW
    nchunk = pts_per_w // B

    mesh = plsc.VectorSubcoreMesh(
        core_axis_name="c", subcore_axis_name="s", num_cores=NC, num_subcores=NS
    )
    scratch = (
        [pltpu.VMEM((B,), jnp.float32),
         pltpu.VMEM((B,), jnp.float32)]
        + [pltpu.VMEM((B,), jnp.int32) for _ in range(2 * NLVL)]
        + [pltpu.VMEM((B, 8), jnp.float32) for _ in range(2 * NLVL)]
        + [pltpu.SemaphoreType.DMA for _ in range(2 * NLVL)]
    )

    @functools.partial(
        pl.kernel,
        out_type=tuple(jax.ShapeDtypeStruct((npts, 8), jnp.float32)
                       for _ in range(2 * NLVL)),
        mesh=mesh,
        scratch_types=scratch,
        compiler_params=pltpu.CompilerParams(use_tc_tiling_on_sc=False),
    )
    def k(x_hbm, t0_hbm, t1_hbm, t2_hbm, o0u, o0d, o1u, o1d, o2u, o2d,
          u_ref, v_ref, i0u, i0d, i1u, i1d, i2u, i2d,
          r0u, r0d, r1u, r1d, r2u, r2d, s0u, s0d, s1u, s1d, s2u, s2d):
        t_hbm = (t0_hbm, t0_hbm, t1_hbm, t1_hbm, t2_hbm, t2_hbm)
        out_hbm = (o0u, o0d, o1u, o1d, o2u, o2d)
        idx_refs = (i0u, i0d, i1u, i1d, i2u, i2d)
        row_refs = (r0u, r0d, r1u, r1d, r2u, r2d)
        sems = (s0u, s0d, s1u, s1d, s2u, s2d)
        wid = lax.axis_index("s") * NC + lax.axis_index("c")
        groups = B // L

        def chunk_body(ci, carry):
            base = wid * pts_per_w + ci * B
            pltpu.sync_copy(x_hbm.at[pl.ds(n + lo + base, B)], u_ref)
            pltpu.sync_copy(x_hbm.at[pl.ds(2 * n + lo + base, B)], v_ref)

            def idx_body(j, c):
                sl = pl.ds(j * L, L)
                u = u_ref[sl]
                v = v_ref[sl]
                for lvl, r in enumerate(RES):
                    x0 = (u * r).astype(jnp.int32)
                    y0 = (v * r).astype(jnp.int32)
                    p = y0 * r + x0
                    up = (p >> 1) + (p & 1) * H[lvl]
                    idx_refs[2 * lvl][sl] = up
                    idx_refs[2 * lvl + 1][sl] = up + (r // 2)
                return c

            lax.fori_loop(0, groups, idx_body, 0)

            dmas = []
            for st in range(2 * NLVL):
                for g in range(NDMA):
                    dmas.append(pltpu.async_copy(
                        t_hbm[st].at[idx_refs[st].at[pl.ds(g * ROWS_PER_DMA, ROWS_PER_DMA)]],
                        row_refs[st].at[pl.ds(g * ROWS_PER_DMA, ROWS_PER_DMA)],
                        sems[st],
                    ))
            for st in range(2 * NLVL):
                for g in range(NDMA):
                    dmas[st * NDMA + g].wait()
                pltpu.sync_copy(row_refs[st], out_hbm[st].at[pl.ds(base, B), :])
            return carry

        lax.fori_loop(0, nchunk, chunk_body, 0)

    return k(xflat, t0, t1, t2)


def _tc_body(u0, d0, u1, d1, u2, d2, u8, v8, i8, eb, g0, g1, g2, gi,
             b0t, w1t, b1r, w2t, b2r, out_ref):
    e = eb[...]
    u64 = jnp.dot(u8[...], e, preferred_element_type=jnp.float32)
    v64 = jnp.dot(v8[...], e, preferred_element_type=jnp.float32)
    cf = jax.lax.broadcasted_iota(jnp.int32, (1, 64), 1) % 8
    is_x1 = cf >= 4
    acc = jnp.dot(i8[...], gi[...], preferred_element_type=jnp.float32)
    ups = (u0, u1, u2)
    dns = (d0, d1, d2)
    gs = (g0, g1, g2)
    for lvl, r in enumerate(RES):
        fx = u64 * r
        wx = fx - jnp.floor(fx)
        fy = v64 * r
        wy = fy - jnp.floor(fy)
        wxs = jnp.where(is_x1, wx, 1.0 - wx)
        t = wxs * (ups[lvl][...] * (1.0 - wy) + dns[lvl][...] * wy)
        acc = acc + jnp.dot(t, gs[lvl][...], preferred_element_type=jnp.float32)
    h1 = jnp.maximum(acc + b0t[...], 0.0)
    outs = []
    for j in range(8):
        hj = h1[:, 64 * j:64 * j + 64]
        h2 = jnp.maximum(
            jnp.dot(hj, w1t[...], preferred_element_type=jnp.float32) + b1r[...], 0.0)
        outs.append(
            jnp.dot(h2, w2t[...], preferred_element_type=jnp.float32) + b2r[...])
    out_ref[...] = jnp.concatenate(outs, axis=1)


def _tc_mlp(c, u8, v8, i8, off, mh, W0, b0, W1, b1, W2, b2):
    mb = 1024
    grid = (mh // mb,)
    eye8 = jnp.eye(8, dtype=jnp.float32)
    eb = jnp.kron(eye8, jnp.ones((1, 8), jnp.float32))
    gi = jnp.kron(eye8, W0[:, 0:1].T)
    gs = []
    for lvl in range(NLVL):
        e8 = jnp.tile(W0[:, 1 + 4 * lvl:5 + 4 * lvl].T, (2, 1))
        gs.append(jnp.kron(eye8, e8))
    b0t = jnp.tile(b0, 8)[None, :]
    row = lambda i: (i, 0)
    rowo = lambda i: (i + off, 0)
    full = lambda i: (0, 0)
    out_dim = W2.shape[0]
    return pl.pallas_call(
        _tc_body,
        grid=grid,
        in_specs=[
            pl.BlockSpec((mb, 64), rowo),
            pl.BlockSpec((mb, 64), rowo),
            pl.BlockSpec((mb, 64), rowo),
            pl.BlockSpec((mb, 64), rowo),
            pl.BlockSpec((mb, 64), rowo),
            pl.BlockSpec((mb, 64), rowo),
            pl.BlockSpec((mb, 8), rowo),
            pl.BlockSpec((mb, 8), rowo),
            pl.BlockSpec((mb, 8), rowo),
            pl.BlockSpec((8, 64), full),
            pl.BlockSpec((64, 512), full),
            pl.BlockSpec((64, 512), full),
            pl.BlockSpec((64, 512), full),
            pl.BlockSpec((8, 512), full),
            pl.BlockSpec((1, 512), full),
            pl.BlockSpec((64, 64), full),
            pl.BlockSpec((1, 64), full),
            pl.BlockSpec((64, out_dim), full),
            pl.BlockSpec((1, out_dim), full),
        ],
        out_specs=pl.BlockSpec((mb, 8 * out_dim), row),
        out_shape=jax.ShapeDtypeStruct((mh, 8 * out_dim), jnp.float32),
    )(*c, u8, v8, i8, eb, gs[0], gs[1], gs[2], gi, b0t,
      W1.T, b1[None, :], W2.T, b2[None, :])


def kernel(x, emb0, emb1, emb2, W0, b0, W1, b1, W2, b2):
    n = x.shape[0]
    nh = n // 2
    xT = x.T
    xflat = xT.reshape(-1)
    tables = [
        _pair_table_big(emb0, H[0], _BQ[0]),
        _pair_table_big(emb1, H[1], _BQ[1]),
        _pair_table_small(emb2, RES[2]),
    ]
    c = _sc_gather(xflat, n, 0, n, *tables)
    m = n // 8
    u8 = xT[1].reshape(m, 8)
    v8 = xT[2].reshape(m, 8)
    i8 = xT[0].reshape(m, 8)
    packed = [a.reshape(m, 64) for a in c]
    out = _tc_mlp(packed, u8, v8, i8, 0, m, W0, b0, W1, b1, W2, b2)
    return out.reshape(n, W2.shape[0])


# speedup vs baseline: 1.0225x; 1.0010x over previous
import functools

import jax
import jax.numpy as jnp
from jax import lax
from jax.experimental import pallas as pl
from jax.experimental.pallas import tpu as pltpu
from jax.experimental.pallas import tpu_sc as plsc

RES = (512, 264, 16)
NLVL = 3
NC, NS, L = 2, 16, 16
NW = NC * NS
B = 2048
ROWS_PER_DMA = 256
NDMA = B // ROWS_PER_DMA

H = (131584, 35072, 144)
_BQ = (2056, 2192)


def _pair_table_small(emb, r):
    s = (r + 1) * (r + 1)
    h = (4 * s) // 8
    flat = emb.reshape(-1)
    return jnp.concatenate(
        [flat[: 8 * h].reshape(h, 8), flat[4 : 4 + 8 * h].reshape(h, 8)], axis=0)


def _table_body(a_ref, b_ref, o_ref):
    a = a_ref[...]

    @pl.when(pl.program_id(0) == 0)
    def _():
        o_ref[...] = a

    @pl.when(pl.program_id(0) == 1)
    def _():
        b = b_ref[...]
        a1 = jnp.concatenate([a[1:], b[:1]], axis=0)
        o_ref[...] = jnp.concatenate([a[:, 4:], a1[:, :4]], axis=1)


def _pair_table_big(emb, h, bq):
    flat = emb.reshape(-1)
    fv = flat.shape[0] // 128
    wide = flat[: fv * 128].reshape(fv, 128)
    qh = h // 16
    nq = qh // bq
    out = pl.pallas_call(
        _table_body,
        grid=(2, nq),
        in_specs=[
            pl.BlockSpec((bq, 128), lambda h2, q: (q, 0)),
            pl.BlockSpec((bq, 128), lambda h2, q: (q + 1, 0)),
        ],
        out_specs=pl.BlockSpec((bq, 128), lambda h2, q: (h2 * (h // 16 // bq) + q, 0)),
        out_shape=jax.ShapeDtypeStruct((2 * qh, 128), jnp.float32),
    )(wide, wide)
    return out.reshape(2 * h, 8)


def _sc_gather(xflat, n, lo, npts, t0, t1, t2):
    pts_per_w = npts // NW
    nchunk = pts_per_w // B

    mesh = plsc.VectorSubcoreMesh(
        core_axis_name="c", subcore_axis_name="s", num_cores=NC, num_subcores=NS
    )
    scratch = (
        [pltpu.VMEM((B,), jnp.float32),
         pltpu.VMEM((B,), jnp.float32)]
        + [pltpu.VMEM((B,), jnp.int32) for _ in range(2 * NLVL)]
        + [pltpu.VMEM((B, 8), jnp.float32) for _ in range(2 * NLVL)]
        + [pltpu.SemaphoreType.DMA for _ in range(2 * NLVL)]
    )

    @functools.partial(
        pl.kernel,
        out_type=tuple(jax.ShapeDtypeStruct((npts, 8), jnp.float32)
                       for _ in range(2 * NLVL)),
        mesh=mesh,
        scratch_types=scratch,
        compiler_params=pltpu.CompilerParams(use_tc_tiling_on_sc=False),
    )
    def k(x_hbm, t0_hbm, t1_hbm, t2_hbm, o0u, o0d, o1u, o1d, o2u, o2d,
          u_ref, v_ref, i0u, i0d, i1u, i1d, i2u, i2d,
          r0u, r0d, r1u, r1d, r2u, r2d, s0u, s0d, s1u, s1d, s2u, s2d):
        t_hbm = (t0_hbm, t0_hbm, t1_hbm, t1_hbm, t2_hbm, t2_hbm)
        out_hbm = (o0u, o0d, o1u, o1d, o2u, o2d)
        idx_refs = (i0u, i0d, i1u, i1d, i2u, i2d)
        row_refs = (r0u, r0d, r1u, r1d, r2u, r2d)
        sems = (s0u, s0d, s1u, s1d, s2u, s2d)
        wid = lax.axis_index("s") * NC + lax.axis_index("c")
        groups = B // L

        def chunk_body(ci, carry):
            base = wid * pts_per_w + ci * B
            pltpu.sync_copy(x_hbm.at[pl.ds(n + lo + base, B)], u_ref)
            pltpu.sync_copy(x_hbm.at[pl.ds(2 * n + lo + base, B)], v_ref)

            def idx_body(j, c):
                sl = pl.ds(j * L, L)
                u = u_ref[sl]
                v = v_ref[sl]
                for lvl, r in enumerate(RES):
                    x0 = (u * r).astype(jnp.int32)
                    y0 = (v * r).astype(jnp.int32)
                    p = y0 * r + x0
                    up = (p >> 1) + (p & 1) * H[lvl]
                    idx_refs[2 * lvl][sl] = up
                    idx_refs[2 * lvl + 1][sl] = up + (r // 2)
                return c

            lax.fori_loop(0, groups, idx_body, 0)

            dmas = []
            for st in range(2 * NLVL):
                for g in range(NDMA):
                    dmas.append(pltpu.async_copy(
                        t_hbm[st].at[idx_refs[st].at[pl.ds(g * ROWS_PER_DMA, ROWS_PER_DMA)]],
                        row_refs[st].at[pl.ds(g * ROWS_PER_DMA, ROWS_PER_DMA)],
                        sems[st],
                    ))
            for st in range(2 * NLVL):
                for g in range(NDMA):
                    dmas[st * NDMA + g].wait()
                pltpu.sync_copy(row_refs[st], out_hbm[st].at[pl.ds(base, B), :])
            return carry

        lax.fori_loop(0, nchunk, chunk_body, 0)

    return k(xflat, t0, t1, t2)


def _tc_body(u0, d0, u1, d1, u2, d2, u8, v8, i8, eb, g0, g1, g2, gi,
             b0t, w1t, b1r, w2t, b2r, out_ref):
    e = eb[...]
    u64 = jnp.dot(u8[...], e, preferred_element_type=jnp.float32)
    v64 = jnp.dot(v8[...], e, preferred_element_type=jnp.float32)
    cf = jax.lax.broadcasted_iota(jnp.int32, (1, 64), 1) % 8
    is_x1 = cf >= 4
    acc = jnp.dot(i8[...], gi[...], preferred_element_type=jnp.float32)
    ups = (u0, u1, u2)
    dns = (d0, d1, d2)
    gs = (g0, g1, g2)
    for lvl, r in enumerate(RES):
        fx = u64 * r
        wx = fx - jnp.floor(fx)
        fy = v64 * r
        wy = fy - jnp.floor(fy)
        wxs = jnp.where(is_x1, wx, 1.0 - wx)
        t = wxs * (ups[lvl][...] * (1.0 - wy) + dns[lvl][...] * wy)
        acc = acc + jnp.dot(t, gs[lvl][...], preferred_element_type=jnp.float32)
    h1 = jnp.maximum(acc + b0t[...], 0.0)
    outs = []
    for j in range(8):
        hj = h1[:, 64 * j:64 * j + 64]
        h2 = jnp.maximum(
            jnp.dot(hj, w1t[...], preferred_element_type=jnp.float32) + b1r[...], 0.0)
        outs.append(
            jnp.dot(h2, w2t[...], preferred_element_type=jnp.float32) + b2r[...])
    out_ref[...] = jnp.concatenate(outs, axis=1)


def _tc_mlp(c, u8, v8, i8, off, mh, W0, b0, W1, b1, W2, b2):
    mb = 1024
    grid = (mh // mb,)
    eye8 = jnp.eye(8, dtype=jnp.float32)
    eb = jnp.kron(eye8, jnp.ones((1, 8), jnp.float32))
    gi = jnp.kron(eye8, W0[:, 0:1].T)
    gs = []
    for lvl in range(NLVL):
        e8 = jnp.tile(W0[:, 1 + 4 * lvl:5 + 4 * lvl].T, (2, 1))
        gs.append(jnp.kron(eye8, e8))
    b0t = jnp.tile(b0, 8)[None, :]
    row = lambda i: (i, 0)
    rowo = lambda i: (i + off, 0)
    full = lambda i: (0, 0)
    out_dim = W2.shape[0]
    return pl.pallas_call(
        _tc_body,
        grid=grid,
        in_specs=[
            pl.BlockSpec((mb, 64), rowo),
            pl.BlockSpec((mb, 64), rowo),
            pl.BlockSpec((mb, 64), rowo),
            pl.BlockSpec((mb, 64), rowo),
            pl.BlockSpec((mb, 64), rowo),
            pl.BlockSpec((mb, 64), rowo),
            pl.BlockSpec((mb, 8), rowo),
            pl.BlockSpec((mb, 8), rowo),
            pl.BlockSpec((mb, 8), rowo),
            pl.BlockSpec((8, 64), full),
            pl.BlockSpec((64, 512), full),
            pl.BlockSpec((64, 512), full),
            pl.BlockSpec((64, 512), full),
            pl.BlockSpec((8, 512), full),
            pl.BlockSpec((1, 512), full),
            pl.BlockSpec((64, 64), full),
            pl.BlockSpec((1, 64), full),
            pl.BlockSpec((64, out_dim), full),
            pl.BlockSpec((1, out_dim), full),
        ],
        out_specs=pl.BlockSpec((mb, 8 * out_dim), row),
        out_shape=jax.ShapeDtypeStruct((mh, 8 * out_dim), jnp.float32),
    )(*c, u8, v8, i8, eb, gs[0], gs[1], gs[2], gi, b0t,
      W1.T, b1[None, :], W2.T, b2[None, :])


def kernel(x, emb0, emb1, emb2, W0, b0, W1, b1, W2, b2):
    n = x.shape[0]
    nh = n // 2
    xT = x.T
    xflat = xT.reshape(-1)
    tables = [
        _pair_table_big(emb0, H[0], _BQ[0]),
        _pair_table_big(emb1, H[1], _BQ[1]),
        _pair_table_small(emb2, RES[2]),
    ]
    c = _sc_gather(xflat, n, 0, n, *tables)
    m = n // 8
    u8 = xT[1].reshape(m, 8)
    v8 = xT[2].reshape(m, 8)
    i8 = xT[0].reshape(m, 8)
    packed = [a.reshape(m, 64) for a in c]
    out = _tc_mlp(packed, u8, v8, i8, 0, m, W0, b0, W1, b1, W2, b2)
    return out.reshape(n, W2.shape[0])


# speedup vs baseline: 1.0242x; 1.0016x over previous
import functools

import jax
import jax.numpy as jnp
from jax import lax
from jax.experimental import pallas as pl
from jax.experimental.pallas import tpu as pltpu
from jax.experimental.pallas import tpu_sc as plsc

RES = (512, 264, 16)
NLVL = 3
NC, NS, L = 2, 16, 16
NW = NC * NS
B = 2048
ROWS_PER_DMA = 512
NDMA = B // ROWS_PER_DMA

H = (131584, 35072, 144)
_BQ = (2056, 2192)


def _pair_table_small(emb, r):
    s = (r + 1) * (r + 1)
    h = (4 * s) // 8
    flat = emb.reshape(-1)
    return jnp.concatenate(
        [flat[: 8 * h].reshape(h, 8), flat[4 : 4 + 8 * h].reshape(h, 8)], axis=0)


def _table_body(a_ref, b_ref, o_ref):
    a = a_ref[...]

    @pl.when(pl.program_id(0) == 0)
    def _():
        o_ref[...] = a

    @pl.when(pl.program_id(0) == 1)
    def _():
        b = b_ref[...]
        a1 = jnp.concatenate([a[1:], b[:1]], axis=0)
        o_ref[...] = jnp.concatenate([a[:, 4:], a1[:, :4]], axis=1)


def _pair_table_big(emb, h, bq):
    flat = emb.reshape(-1)
    fv = flat.shape[0] // 128
    wide = flat[: fv * 128].reshape(fv, 128)
    qh = h // 16
    nq = qh // bq
    out = pl.pallas_call(
        _table_body,
        grid=(2, nq),
        in_specs=[
            pl.BlockSpec((bq, 128), lambda h2, q: (q, 0)),
            pl.BlockSpec((bq, 128), lambda h2, q: (q + 1, 0)),
        ],
        out_specs=pl.BlockSpec((bq, 128), lambda h2, q: (h2 * (h // 16 // bq) + q, 0)),
        out_shape=jax.ShapeDtypeStruct((2 * qh, 128), jnp.float32),
    )(wide, wide)
    return out.reshape(2 * h, 8)


def _sc_gather(xflat, n, lo, npts, t0, t1, t2):
    pts_per_w = npts // NW
    nchunk = pts_per_w // B

    mesh = plsc.VectorSubcoreMesh(
        core_axis_name="c", subcore_axis_name="s", num_cores=NC, num_subcores=NS
    )
    scratch = (
        [pltpu.VMEM((B,), jnp.float32),
         pltpu.VMEM((B,), jnp.float32)]
        + [pltpu.VMEM((B,), jnp.int32) for _ in range(2 * NLVL)]
        + [pltpu.VMEM((B, 8), jnp.float32) for _ in range(2 * NLVL)]
        + [pltpu.SemaphoreType.DMA for _ in range(2 * NLVL)]
    )

    @functools.partial(
        pl.kernel,
        out_type=tuple(jax.ShapeDtypeStruct((npts, 8), jnp.float32)
                       for _ in range(2 * NLVL)),
        mesh=mesh,
        scratch_types=scratch,
        compiler_params=pltpu.CompilerParams(use_tc_tiling_on_sc=False),
    )
    def k(x_hbm, t0_hbm, t1_hbm, t2_hbm, o0u, o0d, o1u, o1d, o2u, o2d,
          u_ref, v_ref, i0u, i0d, i1u, i1d, i2u, i2d,
          r0u, r0d, r1u, r1d, r2u, r2d, s0u, s0d, s1u, s1d, s2u, s2d):
        t_hbm = (t0_hbm, t0_hbm, t1_hbm, t1_hbm, t2_hbm, t2_hbm)
        out_hbm = (o0u, o0d, o1u, o1d, o2u, o2d)
        idx_refs = (i0u, i0d, i1u, i1d, i2u, i2d)
        row_refs = (r0u, r0d, r1u, r1d, r2u, r2d)
        sems = (s0u, s0d, s1u, s1d, s2u, s2d)
        wid = lax.axis_index("s") * NC + lax.axis_index("c")
        groups = B // L

        def chunk_body(ci, carry):
            base = wid * pts_per_w + ci * B
            pltpu.sync_copy(x_hbm.at[pl.ds(n + lo + base, B)], u_ref)
            pltpu.sync_copy(x_hbm.at[pl.ds(2 * n + lo + base, B)], v_ref)

            def idx_body(j, c):
                sl = pl.ds(j * L, L)
                u = u_ref[sl]
                v = v_ref[sl]
                for lvl, r in enumerate(RES):
                    x0 = (u * r).astype(jnp.int32)
                    y0 = (v * r).astype(jnp.int32)
                    p = y0 * r + x0
                    up = (p >> 1) + (p & 1) * H[lvl]
                    idx_refs[2 * lvl][sl] = up
                    idx_refs[2 * lvl + 1][sl] = up + (r // 2)
                return c

            lax.fori_loop(0, groups, idx_body, 0)

            dmas = []
            for st in range(2 * NLVL):
                for g in range(NDMA):
                    dmas.append(pltpu.async_copy(
                        t_hbm[st].at[idx_refs[st].at[pl.ds(g * ROWS_PER_DMA, ROWS_PER_DMA)]],
                        row_refs[st].at[pl.ds(g * ROWS_PER_DMA, ROWS_PER_DMA)],
                        sems[st],
                    ))
            for st in range(2 * NLVL):
                for g in range(NDMA):
                    dmas[st * NDMA + g].wait()
                pltpu.sync_copy(row_refs[st], out_hbm[st].at[pl.ds(base, B), :])
            return carry

        lax.fori_loop(0, nchunk, chunk_body, 0)

    return k(xflat, t0, t1, t2)


def _tc_body(u0, d0, u1, d1, u2, d2, u8, v8, i8, eb, g0, g1, g2, gi,
             b0t, w1t, b1r, w2t, b2r, out_ref):
    e = eb[...]
    u64 = jnp.dot(u8[...], e, preferred_element_type=jnp.float32)
    v64 = jnp.dot(v8[...], e, preferred_element_type=jnp.float32)
    cf = jax.lax.broadcasted_iota(jnp.int32, (1, 64), 1) % 8
    is_x1 = cf >= 4
    acc = jnp.dot(i8[...], gi[...], preferred_element_type=jnp.float32)
    ups = (u0, u1, u2)
    dns = (d0, d1, d2)
    gs = (g0, g1, g2)
    for lvl, r in enumerate(RES):
        fx = u64 * r
        wx = fx - jnp.floor(fx)
        fy = v64 * r
        wy = fy - jnp.floor(fy)
        wxs = jnp.where(is_x1, wx, 1.0 - wx)
        t = wxs * (ups[lvl][...] * (1.0 - wy) + dns[lvl][...] * wy)
        acc = acc + jnp.dot(t, gs[lvl][...], preferred_element_type=jnp.float32)
    h1 = jnp.maximum(acc + b0t[...], 0.0)
    outs = []
    for j in range(8):
        hj = h1[:, 64 * j:64 * j + 64]
        h2 = jnp.maximum(
            jnp.dot(hj, w1t[...], preferred_element_type=jnp.float32) + b1r[...], 0.0)
        outs.append(
            jnp.dot(h2, w2t[...], preferred_element_type=jnp.float32) + b2r[...])
    out_ref[...] = jnp.concatenate(outs, axis=1)


def _tc_mlp(c, u8, v8, i8, off, mh, W0, b0, W1, b1, W2, b2):
    mb = 1024
    grid = (mh // mb,)
    eye8 = jnp.eye(8, dtype=jnp.float32)
    eb = jnp.kron(eye8, jnp.ones((1, 8), jnp.float32))
    gi = jnp.kron(eye8, W0[:, 0:1].T)
    gs = []
    for lvl in range(NLVL):
        e8 = jnp.tile(W0[:, 1 + 4 * lvl:5 + 4 * lvl].T, (2, 1))
        gs.append(jnp.kron(eye8, e8))
    b0t = jnp.tile(b0, 8)[None, :]
    row = lambda i: (i, 0)
    rowo = lambda i: (i + off, 0)
    full = lambda i: (0, 0)
    out_dim = W2.shape[0]
    return pl.pallas_call(
        _tc_body,
        grid=grid,
        in_specs=[
            pl.BlockSpec((mb, 64), rowo),
            pl.BlockSpec((mb, 64), rowo),
            pl.BlockSpec((mb, 64), rowo),
            pl.BlockSpec((mb, 64), rowo),
            pl.BlockSpec((mb, 64), rowo),
            pl.BlockSpec((mb, 64), rowo),
            pl.BlockSpec((mb, 8), rowo),
            pl.BlockSpec((mb, 8), rowo),
            pl.BlockSpec((mb, 8), rowo),
            pl.BlockSpec((8, 64), full),
            pl.BlockSpec((64, 512), full),
            pl.BlockSpec((64, 512), full),
            pl.BlockSpec((64, 512), full),
            pl.BlockSpec((8, 512), full),
            pl.BlockSpec((1, 512), full),
            pl.BlockSpec((64, 64), full),
            pl.BlockSpec((1, 64), full),
            pl.BlockSpec((64, out_dim), full),
            pl.BlockSpec((1, out_dim), full),
        ],
        out_specs=pl.BlockSpec((mb, 8 * out_dim), row),
        out_shape=jax.ShapeDtypeStruct((mh, 8 * out_dim), jnp.float32),
    )(*c, u8, v8, i8, eb, gs[0], gs[1], gs[2], gi, b0t,
      W1.T, b1[None, :], W2.T, b2[None, :])


def kernel(x, emb0, emb1, emb2, W0, b0, W1, b1, W2, b2):
    n = x.shape[0]
    nh = n // 2
    xT = x.T
    xflat = xT.reshape(-1)
    tables = [
        _pair_table_big(emb0, H[0], _BQ[0]),
        _pair_table_big(emb1, H[1], _BQ[1]),
        _pair_table_small(emb2, RES[2]),
    ]
    c = _sc_gather(xflat, n, 0, n, *tables)
    m = n // 8
    u8 = xT[1].reshape(m, 8)
    v8 = xT[2].reshape(m, 8)
    i8 = xT[0].reshape(m, 8)
    packed = [a.reshape(m, 64) for a in c]
    out = _tc_mlp(packed, u8, v8, i8, 0, m, W0, b0, W1, b1, W2, b2)
    return out.reshape(n, W2.shape[0])
